# Initial kernel scaffold; baseline (speedup 1.0000x reference)
import jax
import jax.numpy as jnp
from jax.experimental import pallas as pl


def kernel(x, edge_index_e0, edge_index_e1, edge_index_e2, W_e0, b_e0, W_e1, b_e1, W_e2, b_e2):
    raise NotImplementedError("write your pallas kernel here")


# speedup vs baseline: 5.5031x; 5.5031x over previous
import jax
import jax.numpy as jnp
from jax import lax
from jax.experimental import pallas as pl
from jax.experimental.pallas import tpu as pltpu
from jax.experimental.pallas import tpu_sc as plsc

N = 10000
D = 512
E = 50000
NUM_ETYPES = 3

N_SC = 2
N_TILES = 16
BATCH = 125
NBATCH = 25
EPT = NBATCH * BATCH
STRIPE = 624
TAIL = N - N_TILES * STRIPE


def _sc_body(x0, x1, x2, x3, src_r, dst_r, z128, ones,
             g0, g1, g2, g3, gc,
             acc, sidx, didx, rows, sem):
  c = lax.axis_index("c")
  s = lax.axis_index("s")
  rs = s * STRIPE

  xs = [x0, x1, x2, x3]
  gs = [g0, g1, g2, g3]

  def zero(a, zbuf):
    pltpu.sync_copy(zbuf, a.at[pl.ds(rs, STRIPE)])

    @pl.when(s == N_TILES - 1)
    def _():
      pltpu.sync_copy(zbuf.at[pl.ds(0, TAIL)], a.at[pl.ds(N - TAIL, TAIL)])

  def scatter(xh, a, rbuf, gather):
    @pl.loop(0, NBATCH)
    def _batch(j):
      if gather:
        pltpu.async_copy(xh.at[sidx.at[j]], rbuf, sem).wait()
      pltpu.sync_copy(rbuf, a.at[didx.at[j]], add=True)

  def copyout(e, gh, a):
    pltpu.sync_copy(a.at[pl.ds(rs, STRIPE)], gh.at[e, pl.ds(rs, STRIPE)])

    @pl.when(s == N_TILES - 1)
    def _():
      pltpu.sync_copy(a.at[pl.ds(N - TAIL, TAIL)],
                      gh.at[e, pl.ds(N - TAIL, TAIL)])

  for e in range(NUM_ETYPES):
    pltpu.sync_copy(src_r.at[e, s], sidx)
    pltpu.sync_copy(dst_r.at[e, s], didx)
    for slot in range(3):
      if slot < 2:
        zero(acc, z128)
      else:
        @pl.when(c == 0)
        def _():
          zero(acc, z128)
          pltpu.sync_copy(ones, rows)
      plsc.subcore_barrier()
      if slot < 2:
        for cv in range(N_SC):
          @pl.when(c == cv)
          def _(cv=cv, slot=slot):
            process_chunk = 2 * cv + slot
            scatter(xs[process_chunk], acc, rows, True)
      else:
        @pl.when(c == 0)
        def _():
          scatter(None, acc, rows, False)
      plsc.subcore_barrier()
      if slot < 2:
        for cv in range(N_SC):
          @pl.when(c == cv)
          def _(cv=cv, slot=slot, e=e):
            copyout(e, gs[2 * cv + slot], acc)
      else:
        @pl.when(c == 0)
        def _(e=e):
          copyout(e, gc, acc)


_sc_scatter = pl.kernel(
    _sc_body,
    out_type=[
        jax.ShapeDtypeStruct((NUM_ETYPES, N, 128), jnp.float32),
        jax.ShapeDtypeStruct((NUM_ETYPES, N, 128), jnp.float32),
        jax.ShapeDtypeStruct((NUM_ETYPES, N, 128), jnp.float32),
        jax.ShapeDtypeStruct((NUM_ETYPES, N, 128), jnp.float32),
        jax.ShapeDtypeStruct((NUM_ETYPES, N, 128), jnp.float32),
    ],
    mesh=plsc.VectorSubcoreMesh(core_axis_name="c", subcore_axis_name="s"),
    scratch_types=[
        pltpu.VMEM_SHARED((N, 128), jnp.float32),
        pltpu.VMEM((NBATCH, BATCH), jnp.int32),
        pltpu.VMEM((NBATCH, BATCH), jnp.int32),
        pltpu.VMEM((BATCH, 128), jnp.float32),
        pltpu.SemaphoreType.DMA,
    ],
)


_BN = 400


def _mm_body(g0, g1, g2, g3, gc, w, b, o):
  gs = (g0, g1, g2, g3)
  out = jnp.zeros_like(o)
  for e in range(NUM_ETYPES):
    acc = jnp.zeros_like(o)
    for k in range(4):
      acc += jnp.dot(gs[k][e], w[e, k * 128:(k + 1) * 128, :],
                     preferred_element_type=jnp.float32)
    cnt = gc[e][:, 0:1]
    inv = 1.0 / jnp.maximum(cnt, 1.0)
    mask = (cnt > 0.0).astype(jnp.float32)
    out += acc * inv + mask * b[e]
  o[...] = out


_mm = pl.pallas_call(
    _mm_body,
    grid=(N // _BN,),
    in_specs=[
        pl.BlockSpec((NUM_ETYPES, _BN, 128), lambda r: (0, r, 0)),
        pl.BlockSpec((NUM_ETYPES, _BN, 128), lambda r: (0, r, 0)),
        pl.BlockSpec((NUM_ETYPES, _BN, 128), lambda r: (0, r, 0)),
        pl.BlockSpec((NUM_ETYPES, _BN, 128), lambda r: (0, r, 0)),
        pl.BlockSpec((NUM_ETYPES, _BN, 128), lambda r: (0, r, 0)),
        pl.BlockSpec((NUM_ETYPES, D, D), lambda r: (0, 0, 0)),
        pl.BlockSpec((NUM_ETYPES, 1, D), lambda r: (0, 0, 0)),
    ],
    out_specs=pl.BlockSpec((_BN, D), lambda r: (r, 0)),
    out_shape=jax.ShapeDtypeStruct((N, D), jnp.float32),
    compiler_params=pltpu.CompilerParams(
        dimension_semantics=("parallel",)),
)


@jax.jit
def kernel(x, edge_index_e0, edge_index_e1, edge_index_e2,
           W_e0, b_e0, W_e1, b_e1, W_e2, b_e2):
  xchunks = [x[:, k * 128:(k + 1) * 128] for k in range(4)]

  eis = jnp.stack([edge_index_e0, edge_index_e1, edge_index_e2])
  eis = eis.reshape(NUM_ETYPES, 2, N_TILES, NBATCH, BATCH)
  src_r = eis[:, 0]
  dst_r = eis[:, 1]

  z128 = jnp.zeros((STRIPE, 128), jnp.float32)
  ones = jnp.ones((BATCH, 128), jnp.float32)

  g0, g1, g2, g3, gc = _sc_scatter(*xchunks, src_r, dst_r, z128, ones)

  w = jnp.stack([W_e0, W_e1, W_e2])
  b = jnp.stack([b_e0, b_e1, b_e2]).reshape(NUM_ETYPES, 1, D)
  return _mm(g0, g1, g2, g3, gc, w, b)


# speedup vs baseline: 6.5944x; 1.1983x over previous
import jax
import jax.numpy as jnp
from jax import lax
from jax.experimental import pallas as pl
from jax.experimental.pallas import tpu as pltpu
from jax.experimental.pallas import tpu_sc as plsc

N = 10000
D = 512
E = 50000
NUM_ETYPES = 3

N_SC = 2
N_TILES = 16
BATCH = 125
NBATCH = 25
EPT = NBATCH * BATCH
STRIPE = 624
TAIL = N - N_TILES * STRIPE


def _sc_body(x0, x1, x2, x3, src_r, dst_r, z128, ones,
             g0, g1, g2, g3, gc, gc2,
             acc, sidx, didx, rows0, rows1, gsem, ssem0, ssem1):
  c = lax.axis_index("c")
  s = lax.axis_index("s")
  rs = s * STRIPE

  xs = [x0, x1, x2, x3]
  gs = [g0, g1, g2, g3]

  def zero(a, zbuf):
    pltpu.sync_copy(zbuf, a.at[pl.ds(rs, STRIPE)])

    @pl.when(s == N_TILES - 1)
    def _():
      pltpu.sync_copy(zbuf.at[pl.ds(0, TAIL)], a.at[pl.ds(N - TAIL, TAIL)])

  def scatter(xh, a):
    def gd(j, rb):
      return pltpu.make_async_copy(xh.at[sidx.at[j]], rb, gsem)

    def sd(j, rb, sem):
      return pltpu.make_async_copy(rb, a.at[didx.at[j]], sem)

    def pair(j, first):
      if not first:
        sd(j, rows0, ssem0).wait()
      gd(j, rows0).start()
      gd(j, rows0).wait()
      sd(j, rows0, ssem0).start(add=True)
      if not first:
        sd(j + 1, rows1, ssem1).wait()
      gd(j + 1, rows1).start()
      gd(j + 1, rows1).wait()
      sd(j + 1, rows1, ssem1).start(add=True)

    pair(0, True)

    @pl.loop(1, (NBATCH - 1) // 2)
    def _steady(i):
      pair(2 * i, False)

    j = NBATCH - 1
    sd(j, rows0, ssem0).wait()
    gd(j, rows0).start()
    gd(j, rows0).wait()
    sd(j, rows0, ssem0).start(add=True)
    sd(j, rows1, ssem1).wait()
    sd(j, rows0, ssem0).wait()

  def scatter_ones(a, lo, hi):
    @pl.loop(lo, hi)
    def _fire(j):
      pltpu.make_async_copy(rows0, a.at[didx.at[j]], ssem0).start(add=True)

    @pl.loop(lo, hi)
    def _drain(j):
      pltpu.make_async_copy(rows0, a.at[didx.at[j]], ssem0).wait()

  def copyout(e, gh, a):
    pltpu.sync_copy(a.at[pl.ds(rs, STRIPE)], gh.at[e, pl.ds(rs, STRIPE)])

    @pl.when(s == N_TILES - 1)
    def _():
      pltpu.sync_copy(a.at[pl.ds(N - TAIL, TAIL)],
                      gh.at[e, pl.ds(N - TAIL, TAIL)])

  for e in range(NUM_ETYPES):
    pltpu.sync_copy(src_r.at[e, s], sidx)
    pltpu.sync_copy(dst_r.at[e, s], didx)
    for slot in range(3):
      if slot < 2:
        zero(acc, z128)
        plsc.subcore_barrier()
        for cv in range(N_SC):
          @pl.when(c == cv)
          def _(cv=cv, slot=slot):
            scatter(xs[2 * cv + slot], acc)
        plsc.subcore_barrier()
        for cv in range(N_SC):
          @pl.when(c == cv)
          def _(cv=cv, slot=slot, e=e):
            copyout(e, gs[2 * cv + slot], acc)
      else:
        zero(acc, z128)
        pltpu.sync_copy(ones, rows0)
        plsc.subcore_barrier()
        half = (NBATCH + 1) // 2
        for cv, (lo, hi) in enumerate(((0, half), (half, NBATCH))):
          @pl.when(c == cv)
          def _(lo=lo, hi=hi):
            scatter_ones(acc, lo, hi)
        plsc.subcore_barrier()
        for cv, gh in enumerate((gc, gc2)):
          @pl.when(c == cv)
          def _(gh=gh, e=e):
            copyout(e, gh, acc)


_sc_scatter = pl.kernel(
    _sc_body,
    out_type=[
        jax.ShapeDtypeStruct((NUM_ETYPES, N, 128), jnp.float32),
        jax.ShapeDtypeStruct((NUM_ETYPES, N, 128), jnp.float32),
        jax.ShapeDtypeStruct((NUM_ETYPES, N, 128), jnp.float32),
        jax.ShapeDtypeStruct((NUM_ETYPES, N, 128), jnp.float32),
        jax.ShapeDtypeStruct((NUM_ETYPES, N, 128), jnp.float32),
        jax.ShapeDtypeStruct((NUM_ETYPES, N, 128), jnp.float32),
    ],
    mesh=plsc.VectorSubcoreMesh(core_axis_name="c", subcore_axis_name="s"),
    scratch_types=[
        pltpu.VMEM_SHARED((N, 128), jnp.float32),
        pltpu.VMEM((NBATCH, BATCH), jnp.int32),
        pltpu.VMEM((NBATCH, BATCH), jnp.int32),
        pltpu.VMEM((BATCH, 128), jnp.float32),
        pltpu.VMEM((BATCH, 128), jnp.float32),
        pltpu.SemaphoreType.DMA,
        pltpu.SemaphoreType.DMA,
        pltpu.SemaphoreType.DMA,
    ],
)


_BN = 400


def _mm_body(g0, g1, g2, g3, gc, gc2, w, b, o):
  gs = (g0, g1, g2, g3)
  out = jnp.zeros_like(o)
  for e in range(NUM_ETYPES):
    acc = jnp.zeros_like(o)
    for k in range(4):
      acc += jnp.dot(gs[k][e], w[e, k * 128:(k + 1) * 128, :],
                     preferred_element_type=jnp.float32)
    cnt = gc[e][:, 0:1] + gc2[e][:, 0:1]
    inv = 1.0 / jnp.maximum(cnt, 1.0)
    mask = (cnt > 0.0).astype(jnp.float32)
    out += acc * inv + mask * b[e]
  o[...] = out


_mm = pl.pallas_call(
    _mm_body,
    grid=(N // _BN,),
    in_specs=[
        pl.BlockSpec((NUM_ETYPES, _BN, 128), lambda r: (0, r, 0)),
        pl.BlockSpec((NUM_ETYPES, _BN, 128), lambda r: (0, r, 0)),
        pl.BlockSpec((NUM_ETYPES, _BN, 128), lambda r: (0, r, 0)),
        pl.BlockSpec((NUM_ETYPES, _BN, 128), lambda r: (0, r, 0)),
        pl.BlockSpec((NUM_ETYPES, _BN, 128), lambda r: (0, r, 0)),
        pl.BlockSpec((NUM_ETYPES, _BN, 128), lambda r: (0, r, 0)),
        pl.BlockSpec((NUM_ETYPES, D, D), lambda r: (0, 0, 0)),
        pl.BlockSpec((NUM_ETYPES, 1, D), lambda r: (0, 0, 0)),
    ],
    out_specs=pl.BlockSpec((_BN, D), lambda r: (r, 0)),
    out_shape=jax.ShapeDtypeStruct((N, D), jnp.float32),
    compiler_params=pltpu.CompilerParams(
        dimension_semantics=("parallel",)),
)


@jax.jit
def kernel(x, edge_index_e0, edge_index_e1, edge_index_e2,
           W_e0, b_e0, W_e1, b_e1, W_e2, b_e2):
  xchunks = [x[:, k * 128:(k + 1) * 128] for k in range(4)]

  eis = jnp.stack([edge_index_e0, edge_index_e1, edge_index_e2])
  eis = eis.reshape(NUM_ETYPES, 2, N_TILES, NBATCH, BATCH)
  src_r = eis[:, 0]
  dst_r = eis[:, 1]

  z128 = jnp.zeros((STRIPE, 128), jnp.float32)
  ones = jnp.ones((BATCH, 128), jnp.float32)

  g0, g1, g2, g3, gc, gc2 = _sc_scatter(*xchunks, src_r, dst_r, z128, ones)

  w = jnp.stack([W_e0, W_e1, W_e2])
  b = jnp.stack([b_e0, b_e1, b_e2]).reshape(NUM_ETYPES, 1, D)
  return _mm(g0, g1, g2, g3, gc, gc2, w, b)
